# baseline (speedup 1.0000x reference)
import functools

import jax
import jax.numpy as jnp
from jax import lax
from jax.experimental import pallas as pl
from jax.experimental.pallas import tpu as pltpu
from jax.experimental.pallas import tpu_sc as plsc

_NC = 2
_NS = 16
_NW = _NC * _NS
_L = 16
_ACC = 8

_TC_ROWS = 768


def _tc_rowmax_kernel(x_ref, id_ref, max_ref):
    x = x_ref[...]
    m = jnp.max(x, axis=1, keepdims=True)
    n = x.shape[1]
    iota = lax.broadcasted_iota(jnp.int32, x.shape, 1)
    cand = jnp.where(x == m, iota, jnp.int32(n))
    idx = jnp.min(cand, axis=1, keepdims=True)
    id_ref[...] = idx.astype(jnp.float32)
    max_ref[...] = m


def _tc_rowmax(x2d, rows, block_rows=128):
    n = x2d.shape[1]
    grid = (rows // block_rows,)
    id_out, max_out = pl.pallas_call(
        _tc_rowmax_kernel,
        grid=grid,
        in_specs=[pl.BlockSpec((block_rows, n), lambda i: (i, 0))],
        out_specs=[
            pl.BlockSpec((block_rows, 1), lambda i: (i, 0)),
            pl.BlockSpec((block_rows, 1), lambda i: (i, 0)),
        ],
        out_shape=[
            jax.ShapeDtypeStruct((rows, 1), jnp.float32),
            jax.ShapeDtypeStruct((rows, 1), jnp.float32),
        ],
        compiler_params=pltpu.CompilerParams(
            dimension_semantics=("arbitrary",),
        ),
    )(x2d)
    return id_out[:, 0], max_out[:, 0]


def _lane_gather(x, idx):
    return lax.gather(
        x, idx[:, None],
        lax.GatherDimensionNumbers(
            offset_dims=(), collapsed_slice_dims=(0,), start_index_map=(0,)),
        slice_sizes=(1,),
        mode=lax.GatherScatterMode.PROMISE_IN_BOUNDS)


def _merge(m1, ci1, m2, ci2):
    take2 = (m2 > m1) | ((m2 == m1) & (ci2 < ci1))
    return jnp.where(take2, m2, m1), jnp.where(take2, ci2, ci1)


def _row_reduce(buf, b, n):
    nslice = n // _L
    lane = lax.iota(jnp.int32, _L)

    neg = jnp.full((_L,), -1.0, jnp.float32)
    zero = jnp.zeros((_L,), jnp.int32)
    carry = (neg,) * _ACC + (zero,) * _ACC

    def body(i, c):
        ms, cis = c[:_ACC], c[_ACC:]
        new_ms = []
        new_cis = []
        for k in range(_ACC):
            s = i * _ACC + k
            v = buf[b, pl.ds(s * _L, _L)]
            gt = v > ms[k]
            new_ms.append(jnp.where(gt, v, ms[k]))
            new_cis.append(jnp.where(gt, jnp.full((_L,), s, jnp.int32), cis[k]))
        return tuple(new_ms) + tuple(new_cis)

    out = lax.fori_loop(0, nslice // _ACC, body, carry, unroll=False)
    ms, cis = list(out[:_ACC]), list(out[_ACC:])
    m, ci = ms[0], cis[0]
    for k in range(1, _ACC):
        m, ci = _merge(m, ci, ms[k], cis[k])
    fi = ci * _L + lane
    for k in (8, 4, 2, 1):
        perm = lane ^ k
        m2 = _lane_gather(m, perm)
        fi2 = _lane_gather(fi, perm)
        m, fi = _merge(m, fi, m2, fi2)
    return fi.astype(jnp.float32), m


def _sc_rowmax(row_base, sc_rows, n):
    rows_per_w = sc_rows // _NW
    vrows = max(rows_per_w, _L)
    mesh = plsc.VectorSubcoreMesh(core_axis_name="c", subcore_axis_name="s")

    @functools.partial(
        pl.kernel,
        mesh=mesh,
        out_type=[
            jax.ShapeDtypeStruct((sc_rows,), jnp.float32),
            jax.ShapeDtypeStruct((sc_rows,), jnp.float32),
        ],
        scratch_types=[
            pltpu.VMEM((2, n), jnp.float32),
            pltpu.VMEM((vrows,), jnp.float32),
            pltpu.VMEM((vrows,), jnp.float32),
            pltpu.SemaphoreType.DMA,
            pltpu.SemaphoreType.DMA,
            pltpu.SemaphoreType.DMA,
        ],
    )
    def k(x_hbm, id_hbm, mx_hbm, buf, idv, mxv, sem0, sem1, osem):
        wid = lax.axis_index("s") * _NC + lax.axis_index("c")
        src_base = row_base + wid * rows_per_w
        out_base = wid * rows_per_w
        lane = lax.iota(jnp.int32, _L)
        sems = (sem0, sem1)

        cps = [pltpu.async_copy(x_hbm.at[src_base], buf.at[0], sem0)]
        id_acc = jnp.zeros((_L,), jnp.float32)
        mx_acc = jnp.zeros((_L,), jnp.float32)
        for r in range(rows_per_w):
            b = r % 2
            if r + 1 < rows_per_w:
                cps.append(pltpu.async_copy(
                    x_hbm.at[src_base + r + 1], buf.at[1 - b], sems[1 - b]))
            cps.pop(0).wait()
            idx_f, mx = _row_reduce(buf, b, n)
            sel = lane == (r % _L)
            id_acc = jnp.where(sel, idx_f, id_acc)
            mx_acc = jnp.where(sel, mx, mx_acc)
            if r % _L == _L - 1:
                g0 = r - (_L - 1)
                idv[pl.ds(g0, _L)] = id_acc
                mxv[pl.ds(g0, _L)] = mx_acc
        if rows_per_w % _L != 0:
            g0 = rows_per_w - (rows_per_w % _L)
            idv[pl.ds(g0, _L)] = id_acc
            mxv[pl.ds(g0, _L)] = mx_acc
        pltpu.async_copy(idv.at[pl.ds(0, rows_per_w)],
                         id_hbm.at[pl.ds(out_base, rows_per_w)], osem).wait()
        pltpu.async_copy(mxv.at[pl.ds(0, rows_per_w)],
                         mx_hbm.at[pl.ds(out_base, rows_per_w)], osem).wait()

    return k


@jax.jit
def _rowmax_hybrid(x2d):
    rows, n = x2d.shape
    tc_rows = _TC_ROWS
    sc_rows = rows - tc_rows
    sc_id, sc_mx = _sc_rowmax(tc_rows, sc_rows, n)(x2d)
    tc_id, tc_mx = _tc_rowmax(x2d, tc_rows)
    id_out = jnp.concatenate([tc_id, sc_id])
    mx_out = jnp.concatenate([tc_mx, sc_mx])
    return id_out, mx_out


def kernel(prob_vec):
    b, s, n = prob_vec.shape
    x2d = prob_vec.reshape(b * s, n)
    id_out, max_out = _rowmax_hybrid(x2d)
    out = jnp.stack([id_out, max_out], axis=1)
    return out.reshape(b, s, 2)


# speedup vs baseline: 1.3562x; 1.3562x over previous
import functools

import jax
import jax.numpy as jnp
from jax import lax
from jax.experimental import pallas as pl
from jax.experimental.pallas import tpu as pltpu


def _rowmax_kernel(x_ref, id_ref, max_ref):
    x = x_ref[...]
    m = jnp.max(x, axis=1, keepdims=True)
    n = x.shape[1]
    iota = lax.broadcasted_iota(jnp.int32, x.shape, 1)
    cand = jnp.where(x == m, iota, jnp.int32(n))
    idx = jnp.min(cand, axis=1, keepdims=True)
    id_ref[...] = idx.astype(jnp.float32)
    max_ref[...] = m


@functools.partial(jax.jit, static_argnames=("block_rows",))
def _rowmax(x2d, block_rows=128):
    rows, n = x2d.shape
    grid = (rows // block_rows,)
    id_out, max_out = pl.pallas_call(
        _rowmax_kernel,
        grid=grid,
        in_specs=[pl.BlockSpec((block_rows, n), lambda i: (i, 0))],
        out_specs=[
            pl.BlockSpec((block_rows, 1), lambda i: (i, 0)),
            pl.BlockSpec((block_rows, 1), lambda i: (i, 0)),
        ],
        out_shape=[
            jax.ShapeDtypeStruct((rows, 1), jnp.float32),
            jax.ShapeDtypeStruct((rows, 1), jnp.float32),
        ],
        compiler_params=pltpu.CompilerParams(
            dimension_semantics=("arbitrary",),
        ),
    )(x2d)
    return id_out, max_out


def kernel(prob_vec):
    b, s, n = prob_vec.shape
    x2d = prob_vec.reshape(b * s, n)
    id_out, max_out = _rowmax(x2d)
    out = jnp.concatenate([id_out, max_out], axis=1)
    return out.reshape(b, s, 2)
